# baseline (speedup 1.0000x reference)
import functools

import jax
import jax.numpy as jnp
import numpy as np
from jax import lax
from jax.experimental import pallas as pl
from jax.experimental.pallas import tpu as pltpu
from jax.experimental.pallas import tpu_sc as plsc

_L = 16

_MIN_NORM_IN = np.float32(1e-12)
_MAXNORM = np.float32((1.0 - 1e-15) / np.sqrt(abs(-1.0)))
_MIN_SIZE = np.float32(0.01)
_MAX_SIZE = np.float32(0.999)
_RSQRT_MAGIC = np.int32(0x5F3759DF)


def _rsqrt_newton(ssc):
    i = lax.bitcast_convert_type(ssc, jnp.int32)
    y = lax.bitcast_convert_type(_RSQRT_MAGIC - (i >> 1), jnp.float32)
    h = ssc * jnp.float32(0.5)
    y = y * (jnp.float32(1.5) - h * y * y)
    y = y * (jnp.float32(1.5) - h * y * y)
    return y


def _factors(vs, sclip, sclip_big):
    sq = [v * v for v in vs]
    while len(sq) > 1:
        sq = [sq[i] + sq[i + 1] for i in range(0, len(sq), 2)]
    ss = sq[0]
    ssc = jnp.maximum(ss, jnp.float32(1e-30))
    y = _rsqrt_newton(ssc)
    sqv = ssc * y
    c = jnp.where(sqv > _MIN_NORM_IN, sclip * y, sclip_big)
    ny = c * sqv
    return jnp.where(ny > _MAXNORM, _MAXNORM * y, c)


def _tail_fix(out_t, wt, scale, tail_off, tail_rows, blk):
    n_blk = -(-tail_rows // blk)
    blk0 = tail_off // blk

    def body(w_ref, s_ref, _, o_ref):
        x = w_ref[...]
        sclip = jnp.clip(s_ref[0, 0], _MIN_SIZE, _MAX_SIZE)
        ss = jnp.sum(x * x, axis=0, keepdims=True)
        sqv = jnp.sqrt(ss)
        c = jnp.where(sqv > _MIN_NORM_IN, sclip / sqv,
                      sclip * jnp.float32(1e12))
        ny = c * sqv
        outc = jnp.where(ny > _MAXNORM, _MAXNORM / sqv, c)
        o_ref[...] = x * outc

    return pl.pallas_call(
        body,
        grid=(n_blk,),
        in_specs=[
            pl.BlockSpec((_L, blk), lambda i: (0, blk0 + i)),
            pl.BlockSpec((1, 1), lambda i: (0, 0)),
            pl.BlockSpec(memory_space=pl.ANY),
        ],
        out_specs=pl.BlockSpec((_L, blk), lambda i: (0, blk0 + i)),
        out_shape=jax.ShapeDtypeStruct(out_t.shape, out_t.dtype),
        input_output_aliases={2: 0},
    )(wt, scale.reshape(1, 1), out_t)


_NBUF = 3


def _make_sc_kernel(n_rows, chunk, n_chunks, n_workers):
    mesh = plsc.VectorSubcoreMesh(core_axis_name="c", subcore_axis_name="s")
    num_cores = mesh.num_cores
    nbuf = _NBUF
    max_mine = -(-n_chunks // n_workers)
    loop_hi = -(-max_mine // nbuf) * nbuf

    @functools.partial(
        pl.kernel,
        out_type=jax.ShapeDtypeStruct((_L, n_rows), jnp.float32),
        mesh=mesh,
        scratch_types=(
            [pltpu.VMEM((_L, chunk), jnp.float32)] * (2 * nbuf)
            + [pltpu.VMEM((_L,), jnp.float32)]
            + [pltpu.SemaphoreType.DMA] * (2 * nbuf)
        ),
    )
    def sc_kernel(w_hbm, scale_hbm, out_hbm, *scratch):
        in_bufs = scratch[:nbuf]
        out_bufs = scratch[nbuf:2 * nbuf]
        sbuf = scratch[2 * nbuf]
        in_sems = scratch[2 * nbuf + 1:3 * nbuf + 1]
        out_sems = scratch[3 * nbuf + 1:]

        wid = lax.axis_index("s") * num_cores + lax.axis_index("c")

        pltpu.sync_copy(scale_hbm, sbuf)
        sclip = jnp.clip(sbuf[...], _MIN_SIZE, _MAX_SIZE)
        sclip_big = sclip * jnp.float32(1e12)

        def cols(j):
            idx = j * n_workers + wid
            return pl.ds(pl.multiple_of(idx * chunk, 128), chunk)

        n_mine = jnp.where(wid < (n_chunks % n_workers),
                           n_chunks // n_workers + 1,
                           n_chunks // n_workers).astype(jnp.int32)

        def compute(in_ref, out_ref, lo, hi):
            @plsc.parallel_loop(lo, hi, step=_L, unroll=4)
            def grp(r0):
                sl = pl.ds(r0, _L)
                vs = [in_ref[d, sl] for d in range(_L)]
                outc = _factors(vs, sclip, sclip_big)
                for d in range(_L):
                    out_ref[d, sl] = vs[d] * outc

        def start_in(b, j):
            return pltpu.async_copy(w_hbm.at[:, cols(j)], in_bufs[b],
                                    in_sems[b])

        for b in range(nbuf):
            @pl.when(n_mine > b)
            def _(b=b):
                start_in(b, b)

        def iter_body(j, b):
            @pl.when(j < n_mine)
            def _():
                pltpu.make_async_copy(w_hbm.at[:, cols(j)], in_bufs[b],
                                      in_sems[b]).wait()
                @pl.when(j >= nbuf)
                def _():
                    pltpu.make_async_copy(out_bufs[b],
                                          out_hbm.at[:, cols(j - nbuf)],
                                          out_sems[b]).wait()

                compute(in_bufs[b], out_bufs[b], 0, chunk)
                pltpu.async_copy(out_bufs[b], out_hbm.at[:, cols(j)],
                                 out_sems[b])

                @pl.when(j + nbuf < n_mine)
                def _():
                    start_in(b, j + nbuf)

        @pl.loop(0, loop_hi, step=nbuf)
        def _(j):
            for b in range(nbuf):
                iter_body(j + b, b)

        for b in range(nbuf):
            @pl.when(n_mine > b)
            def _(b=b):
                pltpu.make_async_copy(out_bufs[b],
                                      out_hbm.at[:, pl.ds(0, chunk)],
                                      out_sems[b]).wait()

    return sc_kernel


@jax.jit
def kernel(embeddings_weight, scale):
    n_rows = embeddings_weight.shape[0]
    n_workers = 32
    chunk = 1024
    n_chunks = n_rows // chunk
    tail_off = n_chunks * chunk
    tail_rows = n_rows - tail_off
    scale16 = jnp.broadcast_to(scale.astype(jnp.float32), (_L,))
    wt = embeddings_weight.T
    sck = _make_sc_kernel(n_rows, chunk, n_chunks, n_workers)
    out_t = sck(wt, scale16)
    if tail_rows:
        out_t = _tail_fix(out_t, wt, scale.astype(jnp.float32),
                          tail_off, tail_rows, blk=128)
    return out_t.T


# speedup vs baseline: 1.2765x; 1.2765x over previous
import functools

import jax
import jax.numpy as jnp
import numpy as np
from jax import lax
from jax.experimental import pallas as pl
from jax.experimental.pallas import tpu as pltpu
from jax.experimental.pallas import tpu_sc as plsc

_L = 16

_MIN_NORM_IN = np.float32(1e-12)
_MAXNORM = np.float32((1.0 - 1e-15) / np.sqrt(abs(-1.0)))
_MIN_SIZE = np.float32(0.01)
_MAX_SIZE = np.float32(0.999)
_RSQRT_MAGIC = np.int32(0x5F3759DF)


def _rsqrt_newton(ssc):
    i = lax.bitcast_convert_type(ssc, jnp.int32)
    y = lax.bitcast_convert_type(_RSQRT_MAGIC - (i >> 1), jnp.float32)
    h = ssc * jnp.float32(0.5)
    y = y * (jnp.float32(1.5) - h * y * y)
    y = y * (jnp.float32(1.5) - h * y * y)
    return y


def _factors(vs, sclip, sclip_big):
    sq = [v * v for v in vs]
    while len(sq) > 1:
        sq = [sq[i] + sq[i + 1] for i in range(0, len(sq), 2)]
    ss = sq[0]
    ssc = jnp.maximum(ss, jnp.float32(1e-30))
    y = _rsqrt_newton(ssc)
    sqv = ssc * y
    c = jnp.where(sqv > _MIN_NORM_IN, sclip * y, sclip_big)
    ny = c * sqv
    return jnp.where(ny > _MAXNORM, _MAXNORM * y, c)


def _tail_fix(out_t, wt, scale, tail_off, tail_rows, blk):
    n_blk = -(-tail_rows // blk)
    blk0 = tail_off // blk

    def body(w_ref, s_ref, _, o_ref):
        x = w_ref[...]
        sclip = jnp.clip(s_ref[0, 0], _MIN_SIZE, _MAX_SIZE)
        ss = jnp.sum(x * x, axis=0, keepdims=True)
        sqv = jnp.sqrt(ss)
        c = jnp.where(sqv > _MIN_NORM_IN, sclip / sqv,
                      sclip * jnp.float32(1e12))
        ny = c * sqv
        outc = jnp.where(ny > _MAXNORM, _MAXNORM / sqv, c)
        o_ref[...] = x * outc

    return pl.pallas_call(
        body,
        grid=(n_blk,),
        in_specs=[
            pl.BlockSpec((_L, blk), lambda i: (0, blk0 + i)),
            pl.BlockSpec((1, 1), lambda i: (0, 0)),
            pl.BlockSpec(memory_space=pl.ANY),
        ],
        out_specs=pl.BlockSpec((_L, blk), lambda i: (0, blk0 + i)),
        out_shape=jax.ShapeDtypeStruct(out_t.shape, out_t.dtype),
        input_output_aliases={2: 0},
    )(wt, scale.reshape(1, 1), out_t)


_NBUF = 3


def _make_sc_kernel(n_rows, chunk, n_chunks, n_workers):
    mesh = plsc.VectorSubcoreMesh(core_axis_name="c", subcore_axis_name="s")
    num_cores = mesh.num_cores
    nbuf = _NBUF
    max_mine = -(-n_chunks // n_workers)
    loop_hi = -(-max_mine // nbuf) * nbuf

    @functools.partial(
        pl.kernel,
        out_type=jax.ShapeDtypeStruct((_L, n_rows), jnp.float32),
        mesh=mesh,
        scratch_types=(
            [pltpu.VMEM((_L, chunk), jnp.float32)] * (2 * nbuf)
            + [pltpu.VMEM((_L,), jnp.float32)]
            + [pltpu.SemaphoreType.DMA] * (2 * nbuf)
        ),
    )
    def sc_kernel(w_hbm, scale_hbm, out_hbm, *scratch):
        in_bufs = scratch[:nbuf]
        out_bufs = scratch[nbuf:2 * nbuf]
        sbuf = scratch[2 * nbuf]
        in_sems = scratch[2 * nbuf + 1:3 * nbuf + 1]
        out_sems = scratch[3 * nbuf + 1:]

        wid = lax.axis_index("s") * num_cores + lax.axis_index("c")

        pltpu.sync_copy(scale_hbm, sbuf)
        sclip = jnp.clip(sbuf[...], _MIN_SIZE, _MAX_SIZE)
        sclip_big = sclip * jnp.float32(1e12)

        def cols(j):
            idx = j * n_workers + wid
            return pl.ds(pl.multiple_of(idx * chunk, 128), chunk)

        n_mine = jnp.where(wid < (n_chunks % n_workers),
                           n_chunks // n_workers + 1,
                           n_chunks // n_workers).astype(jnp.int32)

        def compute(in_ref, out_ref, lo, hi):
            @plsc.parallel_loop(lo, hi, step=_L, unroll=2)
            def grp(r0):
                sl = pl.ds(r0, _L)
                vs = [in_ref[d, sl] for d in range(_L)]
                outc = _factors(vs, sclip, sclip_big)
                for d in range(_L):
                    out_ref[d, sl] = vs[d] * outc

        def start_in(b, j):
            return pltpu.async_copy(w_hbm.at[:, cols(j)], in_bufs[b],
                                    in_sems[b])

        for b in range(nbuf):
            @pl.when(n_mine > b)
            def _(b=b):
                start_in(b, b)

        def iter_body(j, b):
            @pl.when(j < n_mine)
            def _():
                pltpu.make_async_copy(w_hbm.at[:, cols(j)], in_bufs[b],
                                      in_sems[b]).wait()
                @pl.when(j >= nbuf)
                def _():
                    pltpu.make_async_copy(out_bufs[b],
                                          out_hbm.at[:, cols(j - nbuf)],
                                          out_sems[b]).wait()

                compute(in_bufs[b], out_bufs[b], 0, chunk)
                pltpu.async_copy(out_bufs[b], out_hbm.at[:, cols(j)],
                                 out_sems[b])

                @pl.when(j + nbuf < n_mine)
                def _():
                    start_in(b, j + nbuf)

        @pl.loop(0, loop_hi, step=nbuf)
        def _(j):
            for b in range(nbuf):
                iter_body(j + b, b)

        for b in range(nbuf):
            @pl.when(n_mine > b)
            def _(b=b):
                pltpu.make_async_copy(out_bufs[b],
                                      out_hbm.at[:, pl.ds(0, chunk)],
                                      out_sems[b]).wait()

    return sc_kernel


@jax.jit
def kernel(embeddings_weight, scale):
    n_rows = embeddings_weight.shape[0]
    n_workers = 32
    chunk = 1024
    n_chunks = n_rows // chunk
    tail_off = n_chunks * chunk
    tail_rows = n_rows - tail_off
    scale16 = jnp.broadcast_to(scale.astype(jnp.float32), (_L,))
    wt = embeddings_weight.T
    sck = _make_sc_kernel(n_rows, chunk, n_chunks, n_workers)
    out_t = sck(wt, scale16)
    if tail_rows:
        out_t = _tail_fix(out_t, wt, scale.astype(jnp.float32),
                          tail_off, tail_rows, blk=128)
    return out_t.T
